# baseline (speedup 1.0000x reference)
import functools

import jax
import jax.numpy as jnp
from jax import lax
from jax.experimental import pallas as pl
from jax.experimental.pallas import tpu as pltpu
from jax.experimental.pallas import tpu_sc as plsc

N = 10000
D = 128
E = 320000

NC = 2
NS = 16
NW = NC * NS

K = 128
EPT = -(-E // NW)
CHH = 2 * (-(-EPT // (4 * K)))
CH = 2 * CHH
E_TILE = CH * K
N_PAD = 10240
SCRAP = N
SLAB = N_PAD // NS
CW = 16

_mesh = plsc.VectorSubcoreMesh(core_axis_name="c", subcore_axis_name="s")


def _make_sc_body(with_count):

    def body(h_hbm, src_hbm, dst_hbm, zeros_hbm, ones_hbm, sums_hbm,
             cnt_hbm, src_v, dst_v, rows0, rows1, sem0, sem1, acc):
        c = lax.axis_index("c")
        s = lax.axis_index("s")
        wid = s * NC + c
        slab = pl.ds(s * SLAB, SLAB)
        pltpu.sync_copy(zeros_hbm.at[slab], acc.at[slab])

        if with_count:
            pltpu.sync_copy(ones_hbm, rows0)
            for hb in range(2):
                pltpu.sync_copy(dst_hbm.at[wid, hb], dst_v)
                plsc.subcore_barrier()

                def cbody(j, carry):
                    pltpu.sync_copy(rows0, acc.at[dst_v.at[j]], add=True)
                    return carry

                lax.fori_loop(0, CHH, cbody, 0)
            plsc.subcore_barrier()
            pltpu.sync_copy(acc.at[slab], cnt_hbm.at[c, slab])
            pltpu.sync_copy(zeros_hbm.at[slab], acc.at[slab])
        plsc.subcore_barrier()

        def gather(j, buf, sem):
            pltpu.async_copy(h_hbm.at[src_v.at[j]], buf, sem)

        def gwait(buf, sem):
            pltpu.make_async_copy(h_hbm.at[src_v.at[0]], buf, sem).wait()

        for hb in range(2):
            pltpu.sync_copy(src_hbm.at[wid, hb], src_v)
            pltpu.sync_copy(dst_hbm.at[wid, hb], dst_v)
            gather(0, rows0, sem0)

            def pbody(i, carry):
                j0 = 2 * i
                gather(j0 + 1, rows1, sem1)
                gwait(rows0, sem0)
                pltpu.sync_copy(rows0, acc.at[dst_v.at[j0]], add=True)
                gather(j0 + 2, rows0, sem0)
                gwait(rows1, sem1)
                pltpu.sync_copy(rows1, acc.at[dst_v.at[j0 + 1]], add=True)
                return carry

            lax.fori_loop(0, CHH // 2, pbody, 0)
            gwait(rows0, sem0)
        plsc.subcore_barrier()
        pltpu.sync_copy(acc.at[slab], sums_hbm.at[c, slab])

    return body


_sc_out = jax.ShapeDtypeStruct((NC, N_PAD, D), jnp.float32)
_sc_scratch = [
    pltpu.VMEM((CHH + 1, K), jnp.int32),
    pltpu.VMEM((CHH, K), jnp.int32),
    pltpu.VMEM((K, D), jnp.float32),
    pltpu.VMEM((K, D), jnp.float32),
    pltpu.SemaphoreType.DMA,
    pltpu.SemaphoreType.DMA,
    pltpu.VMEM_SHARED((N_PAD, D), jnp.float32),
]

_sc_scatter_cnt = pl.kernel(
    _make_sc_body(True),
    out_type=(_sc_out, _sc_out),
    mesh=_mesh,
    scratch_types=_sc_scratch,
)


def _sc_scatter_nc_body(h_hbm, src_hbm, dst_hbm, zeros_hbm, sums_hbm,
                        src_v, dst_v, rows0, rows1, sem0, sem1, acc):
    return _make_sc_body(False)(h_hbm, src_hbm, dst_hbm, zeros_hbm, None,
                                sums_hbm, None, src_v, dst_v, rows0, rows1,
                                sem0, sem1, acc)


_sc_scatter = pl.kernel(
    _sc_scatter_nc_body,
    out_type=_sc_out,
    mesh=_mesh,
    scratch_types=_sc_scratch,
)


BR = 512


def _tc_dense_body(sums_ref, h_ref, cnt_ref, wa_ref, wh_ref, b_ref, out_ref):
    ssum = sums_ref[0] + sums_ref[1]
    cnt = cnt_ref[0][:, :1] + cnt_ref[1][:, :1]
    inv = 1.0 / jnp.maximum(cnt, 1.0)
    agg = jnp.dot(ssum, wa_ref[...], preferred_element_type=jnp.float32) * inv
    o = agg + jnp.dot(h_ref[...], wh_ref[...],
                      preferred_element_type=jnp.float32) + b_ref[...]
    out_ref[...] = jnp.tanh(o)


_tc_dense = pl.pallas_call(
    _tc_dense_body,
    grid=(N_PAD // BR,),
    in_specs=[
        pl.BlockSpec((NC, BR, D), lambda i: (0, i, 0)),
        pl.BlockSpec((BR, D), lambda i: (i, 0)),
        pl.BlockSpec((NC, BR, D), lambda i: (0, i, 0)),
        pl.BlockSpec((D, D), lambda i: (0, 0)),
        pl.BlockSpec((D, D), lambda i: (0, 0)),
        pl.BlockSpec((1, D), lambda i: (0, 0)),
    ],
    out_specs=pl.BlockSpec((BR, D), lambda i: (i, 0)),
    out_shape=jax.ShapeDtypeStruct((N_PAD, D), jnp.float32),
)


def kernel(x, edge_index, W1, b1, W2, b2, W3, b3):
    src = edge_index[0].astype(jnp.int32)
    dst = edge_index[1].astype(jnp.int32)
    pad = NW * E_TILE - E
    src3 = jnp.concatenate([src, jnp.zeros((pad,), jnp.int32)]).reshape(
        NW, 2, CHH, K)
    src3 = jnp.concatenate([src3, jnp.zeros((NW, 2, 1, K), jnp.int32)],
                           axis=2)
    dst3 = jnp.concatenate([dst, jnp.full((pad,), SCRAP, jnp.int32)]).reshape(
        NW, 2, CHH, K)
    zeros = jnp.zeros((N_PAD, D), jnp.float32)
    ones_c = jnp.ones((K, D), jnp.float32)
    h = jnp.zeros((N_PAD, D), jnp.float32).at[:N].set(x)

    for i, (W, b) in enumerate(((W1, b1), (W2, b2), (W3, b3))):
        waT = W[:, :D].T
        whT = W[:, D:].T
        if i == 0:
            sums2, cnt2 = _sc_scatter_cnt(h, src3, dst3, zeros, ones_c)
        else:
            sums2 = _sc_scatter(h, src3, dst3, zeros)
        h = _tc_dense(sums2, h, cnt2, waT, whT, b.reshape(1, D))
    return h[:N]


# speedup vs baseline: 1.3814x; 1.3814x over previous
import functools

import jax
import jax.numpy as jnp
from jax import lax
from jax.experimental import pallas as pl
from jax.experimental.pallas import tpu as pltpu
from jax.experimental.pallas import tpu_sc as plsc

N = 10000
D = 128
E = 320000

NC = 2
NS = 16
NW = NC * NS

K = 128
EPT = -(-E // NW)
CHH = 2 * (-(-EPT // (4 * K)))
CH = 2 * CHH
E_TILE = CH * K
N_PAD = 10240
SCRAP = N
SLAB = N_PAD // NS
CW = 16

_mesh = plsc.VectorSubcoreMesh(core_axis_name="c", subcore_axis_name="s")


def _sc_scatter_body(h_hbm, src_hbm, dst_hbm, zeros_hbm, sums_hbm,
                     src_v, dst_v, rows_v, sem, acc):
    c = lax.axis_index("c")
    s = lax.axis_index("s")
    wid = s * NC + c
    slab = pl.ds(s * SLAB, SLAB)
    pltpu.sync_copy(zeros_hbm.at[slab], acc.at[slab])
    plsc.subcore_barrier()
    for hb in range(2):
        pltpu.sync_copy(src_hbm.at[wid, hb], src_v)
        pltpu.sync_copy(dst_hbm.at[wid, hb], dst_v)

        def pbody(j, carry):
            pltpu.async_copy(h_hbm.at[src_v.at[j]], rows_v, sem).wait()
            pltpu.sync_copy(rows_v, acc.at[dst_v.at[j]], add=True)
            return carry

        lax.fori_loop(0, CHH, pbody, 0)
    plsc.subcore_barrier()
    pltpu.sync_copy(acc.at[slab], sums_hbm.at[c, slab])


_sc_out = jax.ShapeDtypeStruct((NC, N_PAD, D), jnp.float32)

_sc_scatter = pl.kernel(
    _sc_scatter_body,
    out_type=_sc_out,
    mesh=_mesh,
    scratch_types=[
        pltpu.VMEM((CHH, K), jnp.int32),
        pltpu.VMEM((CHH, K), jnp.int32),
        pltpu.VMEM((K, D), jnp.float32),
        pltpu.SemaphoreType.DMA,
        pltpu.VMEM_SHARED((N_PAD, D), jnp.float32),
    ],
)


def _sc_count_body(dst_hbm, zeros_hbm, ones_hbm, cnt_hbm, dst_v, ones_v,
                   acc):
    c = lax.axis_index("c")
    s = lax.axis_index("s")
    wid = s * NC + c
    slab = pl.ds(s * SLAB, SLAB)
    pltpu.sync_copy(zeros_hbm.at[slab], acc.at[slab])
    pltpu.sync_copy(ones_hbm, ones_v)
    plsc.subcore_barrier()
    for hb in range(2):
        pltpu.sync_copy(dst_hbm.at[wid, hb], dst_v)

        def cbody(j, carry):
            pltpu.sync_copy(ones_v, acc.at[dst_v.at[j]], add=True)
            return carry

        lax.fori_loop(0, CHH, cbody, 0)
    plsc.subcore_barrier()
    pltpu.sync_copy(acc.at[slab], cnt_hbm.at[c, slab])


_sc_count = pl.kernel(
    _sc_count_body,
    out_type=_sc_out,
    mesh=_mesh,
    scratch_types=[
        pltpu.VMEM((CHH, K), jnp.int32),
        pltpu.VMEM((K, D), jnp.float32),
        pltpu.VMEM_SHARED((N_PAD, D), jnp.float32),
    ],
)


BR = 512


def _tc_dense_body(sums_ref, h_ref, cnt_ref, wa_ref, wh_ref, b_ref, out_ref):
    ssum = sums_ref[0] + sums_ref[1]
    cnt = cnt_ref[0][:, :1] + cnt_ref[1][:, :1]
    inv = 1.0 / jnp.maximum(cnt, 1.0)
    agg = jnp.dot(ssum, wa_ref[...], preferred_element_type=jnp.float32) * inv
    o = agg + jnp.dot(h_ref[...], wh_ref[...],
                      preferred_element_type=jnp.float32) + b_ref[...]
    out_ref[...] = jnp.tanh(o)


_tc_dense = pl.pallas_call(
    _tc_dense_body,
    grid=(N_PAD // BR,),
    in_specs=[
        pl.BlockSpec((NC, BR, D), lambda i: (0, i, 0)),
        pl.BlockSpec((BR, D), lambda i: (i, 0)),
        pl.BlockSpec((NC, BR, D), lambda i: (0, i, 0)),
        pl.BlockSpec((D, D), lambda i: (0, 0)),
        pl.BlockSpec((D, D), lambda i: (0, 0)),
        pl.BlockSpec((1, D), lambda i: (0, 0)),
    ],
    out_specs=pl.BlockSpec((BR, D), lambda i: (i, 0)),
    out_shape=jax.ShapeDtypeStruct((N_PAD, D), jnp.float32),
)


def kernel(x, edge_index, W1, b1, W2, b2, W3, b3):
    src = edge_index[0].astype(jnp.int32)
    dst = edge_index[1].astype(jnp.int32)
    pad = NW * E_TILE - E
    src3 = jnp.concatenate([src, jnp.zeros((pad,), jnp.int32)]).reshape(
        NW, 2, CHH, K)
    dst3 = jnp.concatenate([dst, jnp.full((pad,), SCRAP, jnp.int32)]).reshape(
        NW, 2, CHH, K)
    zeros = jnp.zeros((N_PAD, D), jnp.float32)
    ones_c = jnp.ones((K, D), jnp.float32)
    h = jnp.zeros((N_PAD, D), jnp.float32).at[:N].set(x)

    cnt2 = _sc_count(dst3, zeros, ones_c)
    for W, b in ((W1, b1), (W2, b2), (W3, b3)):
        waT = W[:, :D].T
        whT = W[:, D:].T
        sums2 = _sc_scatter(h, src3, dst3, zeros)
        h = _tc_dense(sums2, h, cnt2, waT, whT, b.reshape(1, D))
    return h[:N]


# speedup vs baseline: 2.0724x; 1.5002x over previous
import functools

import jax
import jax.numpy as jnp
from jax import lax
from jax.experimental import pallas as pl
from jax.experimental.pallas import tpu as pltpu
from jax.experimental.pallas import tpu_sc as plsc

N = 10000
D = 128
E = 320000

NC = 2
NS = 16
NW = NC * NS

K = 128
EPT = -(-E // NW)
CH = -(-EPT // K)
E_TILE = CH * K
N_PAD = 10240
SCRAP = N
SLAB = N_PAD // NS
CW = 16

_mesh = plsc.VectorSubcoreMesh(core_axis_name="c", subcore_axis_name="s")


def _sc_scatter_body(h_hbm, src_hbm, dst_hbm, zeros_hbm, sums_hbm,
                     src_v, dst_v, rows_v, sem, acc):
    c = lax.axis_index("c")
    s = lax.axis_index("s")
    wid = s * NC + c
    slab = pl.ds(s * SLAB, SLAB)
    pltpu.sync_copy(zeros_hbm.at[slab], acc.at[slab])
    pltpu.sync_copy(src_hbm.at[wid], src_v)
    pltpu.sync_copy(dst_hbm.at[wid], dst_v)
    plsc.subcore_barrier()

    def pbody(j, carry):
        pltpu.async_copy(h_hbm.at[src_v.at[j]], rows_v, sem).wait()
        pltpu.sync_copy(rows_v, acc.at[dst_v.at[j]], add=True)
        return carry

    lax.fori_loop(0, CH, pbody, 0)
    plsc.subcore_barrier()
    pltpu.sync_copy(acc.at[slab], sums_hbm.at[c, slab])


_sc_out = jax.ShapeDtypeStruct((NC, N_PAD, D), jnp.float32)

_sc_scatter = pl.kernel(
    _sc_scatter_body,
    out_type=_sc_out,
    mesh=_mesh,
    scratch_types=[
        pltpu.VMEM((CH, K), jnp.int32),
        pltpu.VMEM((CH, K), jnp.int32),
        pltpu.VMEM((K, D), jnp.float32),
        pltpu.SemaphoreType.DMA,
        pltpu.VMEM_SHARED((N_PAD, D), jnp.float32),
    ],
)


def _sc_count_body(dst_hbm, zeros_hbm, ones_hbm, cnt_hbm, dst_v, ones_v,
                   acc):
    c = lax.axis_index("c")
    s = lax.axis_index("s")
    wid = s * NC + c
    slab = pl.ds(s * SLAB, SLAB)
    pltpu.sync_copy(zeros_hbm.at[slab], acc.at[slab])
    pltpu.sync_copy(ones_hbm, ones_v)
    pltpu.sync_copy(dst_hbm.at[wid], dst_v)
    plsc.subcore_barrier()

    def cbody(j, carry):
        pltpu.sync_copy(ones_v, acc.at[dst_v.at[j]], add=True)
        return carry

    lax.fori_loop(0, CH, cbody, 0)
    plsc.subcore_barrier()
    pltpu.sync_copy(acc.at[slab], cnt_hbm.at[c, slab])


_sc_count = pl.kernel(
    _sc_count_body,
    out_type=_sc_out,
    mesh=_mesh,
    scratch_types=[
        pltpu.VMEM((CH, K), jnp.int32),
        pltpu.VMEM((K, D), jnp.float32),
        pltpu.VMEM_SHARED((N_PAD, D), jnp.float32),
    ],
)


BR = 512


def _tc_dense_body(sums_ref, h_ref, cnt_ref, wa_ref, wh_ref, b_ref, out_ref):
    ssum = sums_ref[0] + sums_ref[1]
    cnt = cnt_ref[0][:, :1] + cnt_ref[1][:, :1]
    inv = 1.0 / jnp.maximum(cnt, 1.0)
    agg = jnp.dot(ssum, wa_ref[...], preferred_element_type=jnp.float32) * inv
    o = agg + jnp.dot(h_ref[...], wh_ref[...],
                      preferred_element_type=jnp.float32) + b_ref[...]
    out_ref[...] = jnp.tanh(o)


_tc_dense = pl.pallas_call(
    _tc_dense_body,
    grid=(N_PAD // BR,),
    in_specs=[
        pl.BlockSpec((NC, BR, D), lambda i: (0, i, 0)),
        pl.BlockSpec((BR, D), lambda i: (i, 0)),
        pl.BlockSpec((NC, BR, D), lambda i: (0, i, 0)),
        pl.BlockSpec((D, D), lambda i: (0, 0)),
        pl.BlockSpec((D, D), lambda i: (0, 0)),
        pl.BlockSpec((1, D), lambda i: (0, 0)),
    ],
    out_specs=pl.BlockSpec((BR, D), lambda i: (i, 0)),
    out_shape=jax.ShapeDtypeStruct((N_PAD, D), jnp.float32),
)


def kernel(x, edge_index, W1, b1, W2, b2, W3, b3):
    src = edge_index[0].astype(jnp.int32)
    dst = edge_index[1].astype(jnp.int32)
    pad = NW * E_TILE - E
    src3 = jnp.concatenate([src, jnp.zeros((pad,), jnp.int32)]).reshape(
        NW, CH, K)
    dst3 = jnp.concatenate([dst, jnp.full((pad,), SCRAP, jnp.int32)]).reshape(
        NW, CH, K)
    zeros = jnp.zeros((N_PAD, D), jnp.float32)
    ones_c = jnp.ones((K, D), jnp.float32)
    h = jnp.zeros((N_PAD, D), jnp.float32).at[:N].set(x)

    cnt2 = _sc_count(dst3, zeros, ones_c)
    for W, b in ((W1, b1), (W2, b2), (W3, b3)):
        waT = W[:, :D].T
        whT = W[:, D:].T
        sums2 = _sc_scatter(h, src3, dst3, zeros)
        h = _tc_dense(sums2, h, cnt2, waT, whT, b.reshape(1, D))
    return h[:N]


# speedup vs baseline: 2.7676x; 1.3355x over previous
import functools

import jax
import jax.numpy as jnp
from jax import lax
from jax.experimental import pallas as pl
from jax.experimental.pallas import tpu as pltpu
from jax.experimental.pallas import tpu_sc as plsc

N = 10000
D = 128
E = 320000

NC = 2
NS = 16
NW = NC * NS

K = 128
BIG_CORE = 0
CH_BIG = 104
CH_SMALL = 53
E_TILE_PAIR = (CH_BIG + CH_SMALL) * K
N_PAD = 10240
SCRAP = N
SLAB = N_PAD // NS
CW = 16

_mesh = plsc.VectorSubcoreMesh(core_axis_name="c", subcore_axis_name="s")


def _sc_scatter_body(h_hbm, src_hbm, dst_hbm, zeros_hbm, sums_hbm,
                     src_v, dst_v, rows_v, sem, acc):
    c = lax.axis_index("c")
    s = lax.axis_index("s")
    wid = s * NC + c
    mych = jnp.where(c == BIG_CORE, CH_BIG, CH_SMALL)
    slab = pl.ds(s * SLAB, SLAB)
    pltpu.sync_copy(zeros_hbm.at[slab], acc.at[slab])
    pltpu.sync_copy(src_hbm.at[wid], src_v)
    pltpu.sync_copy(dst_hbm.at[wid], dst_v)
    plsc.subcore_barrier()

    def pbody(j, carry):
        pltpu.async_copy(h_hbm.at[src_v.at[j]], rows_v, sem).wait()
        pltpu.sync_copy(rows_v, acc.at[dst_v.at[j]], add=True)
        return carry

    lax.fori_loop(0, mych, pbody, 0)
    plsc.subcore_barrier()
    pltpu.sync_copy(acc.at[slab], sums_hbm.at[c, slab])


_sc_out = jax.ShapeDtypeStruct((NC, N_PAD, D), jnp.float32)

_sc_scatter = pl.kernel(
    _sc_scatter_body,
    out_type=_sc_out,
    mesh=_mesh,
    scratch_types=[
        pltpu.VMEM((CH_BIG, K), jnp.int32),
        pltpu.VMEM((CH_BIG, K), jnp.int32),
        pltpu.VMEM((K, D), jnp.float32),
        pltpu.SemaphoreType.DMA,
        pltpu.VMEM_SHARED((N_PAD, D), jnp.float32),
    ],
)


def _sc_count_body(dst_hbm, zeros_hbm, ones_hbm, cnt_hbm, dst_v, ones_v,
                   acc):
    c = lax.axis_index("c")
    s = lax.axis_index("s")
    wid = s * NC + c
    mych = jnp.where(c == BIG_CORE, CH_BIG, CH_SMALL)
    slab = pl.ds(s * SLAB, SLAB)
    pltpu.sync_copy(zeros_hbm.at[slab], acc.at[slab])
    pltpu.sync_copy(ones_hbm, ones_v)
    pltpu.sync_copy(dst_hbm.at[wid], dst_v)
    plsc.subcore_barrier()

    def cbody(j, carry):
        pltpu.sync_copy(ones_v, acc.at[dst_v.at[j]], add=True)
        return carry

    lax.fori_loop(0, mych, cbody, 0)
    plsc.subcore_barrier()
    pltpu.sync_copy(acc.at[slab], cnt_hbm.at[c, slab])


_sc_count = pl.kernel(
    _sc_count_body,
    out_type=_sc_out,
    mesh=_mesh,
    scratch_types=[
        pltpu.VMEM((CH_BIG, K), jnp.int32),
        pltpu.VMEM((K, D), jnp.float32),
        pltpu.VMEM_SHARED((N_PAD, D), jnp.float32),
    ],
)


BR = 512


def _tc_dense_body(sums_ref, h_ref, cnt_ref, wa_ref, wh_ref, b_ref, out_ref):
    ssum = sums_ref[0] + sums_ref[1]
    cnt = cnt_ref[0][:, :1] + cnt_ref[1][:, :1]
    inv = 1.0 / jnp.maximum(cnt, 1.0)
    agg = jnp.dot(ssum, wa_ref[...], preferred_element_type=jnp.float32) * inv
    o = agg + jnp.dot(h_ref[...], wh_ref[...],
                      preferred_element_type=jnp.float32) + b_ref[...]
    out_ref[...] = jnp.tanh(o)


_tc_dense = pl.pallas_call(
    _tc_dense_body,
    grid=(N_PAD // BR,),
    in_specs=[
        pl.BlockSpec((NC, BR, D), lambda i: (0, i, 0)),
        pl.BlockSpec((BR, D), lambda i: (i, 0)),
        pl.BlockSpec((NC, BR, D), lambda i: (0, i, 0)),
        pl.BlockSpec((D, D), lambda i: (0, 0)),
        pl.BlockSpec((D, D), lambda i: (0, 0)),
        pl.BlockSpec((1, D), lambda i: (0, 0)),
    ],
    out_specs=pl.BlockSpec((BR, D), lambda i: (i, 0)),
    out_shape=jax.ShapeDtypeStruct((N_PAD, D), jnp.float32),
)


def kernel(x, edge_index, W1, b1, W2, b2, W3, b3):
    src = edge_index[0].astype(jnp.int32)
    dst = edge_index[1].astype(jnp.int32)
    pad = NS * E_TILE_PAIR - E
    cap_big = NS * CH_BIG * K

    def _layout(flat):
        big = flat[:cap_big].reshape(NS, CH_BIG, K)
        small = flat[cap_big:].reshape(NS, CH_SMALL, K)
        small = jnp.concatenate(
            [small, jnp.zeros((NS, CH_BIG - CH_SMALL, K), jnp.int32)], axis=1)
        pair = (big, small) if BIG_CORE == 0 else (small, big)
        return jnp.stack(pair, axis=1).reshape(NW, CH_BIG, K)

    src3 = _layout(jnp.concatenate([src, jnp.zeros((pad,), jnp.int32)]))
    dst3 = _layout(jnp.concatenate([dst, jnp.full((pad,), SCRAP, jnp.int32)]))
    zeros = jnp.zeros((N_PAD, D), jnp.float32)
    ones_c = jnp.ones((K, D), jnp.float32)
    h = jnp.zeros((N_PAD, D), jnp.float32).at[:N].set(x)

    cnt2 = _sc_count(dst3, zeros, ones_c)
    for W, b in ((W1, b1), (W2, b2), (W3, b3)):
        waT = W[:, :D].T
        whT = W[:, D:].T
        sums2 = _sc_scatter(h, src3, dst3, zeros)
        h = _tc_dense(sums2, h, cnt2, waT, whT, b.reshape(1, D))
    return h[:N]


# speedup vs baseline: 2.9472x; 1.0649x over previous
import functools

import jax
import jax.numpy as jnp
from jax import lax
from jax.experimental import pallas as pl
from jax.experimental.pallas import tpu as pltpu
from jax.experimental.pallas import tpu_sc as plsc

N = 10000
D = 128
E = 320000

NC = 2
NS = 16
NW = NC * NS

K = 128
BIG_CORE = 0
CH_BIG = 96
CH_SMALL = 61
E_TILE_PAIR = (CH_BIG + CH_SMALL) * K
N_PAD = 10240
SCRAP = N
SLAB = N_PAD // NS
CW = 16

_mesh = plsc.VectorSubcoreMesh(core_axis_name="c", subcore_axis_name="s")


def _sc_scatter_body(h_hbm, src_hbm, dst_hbm, zeros_hbm, sums_hbm,
                     src_v, dst_v, rows_v, sem, acc):
    c = lax.axis_index("c")
    s = lax.axis_index("s")
    wid = s * NC + c
    mych = jnp.where(c == BIG_CORE, CH_BIG, CH_SMALL)
    slab = pl.ds(s * SLAB, SLAB)
    pltpu.sync_copy(zeros_hbm.at[slab], acc.at[slab])
    pltpu.sync_copy(src_hbm.at[wid], src_v)
    pltpu.sync_copy(dst_hbm.at[wid], dst_v)
    plsc.subcore_barrier()

    def pbody(j, carry):
        pltpu.async_copy(h_hbm.at[src_v.at[j]], rows_v, sem).wait()
        pltpu.sync_copy(rows_v, acc.at[dst_v.at[j]], add=True)
        return carry

    lax.fori_loop(0, mych, pbody, 0)
    plsc.subcore_barrier()
    pltpu.sync_copy(acc.at[slab], sums_hbm.at[c, slab])


_sc_out = jax.ShapeDtypeStruct((NC, N_PAD, D), jnp.float32)

_sc_scatter = pl.kernel(
    _sc_scatter_body,
    out_type=_sc_out,
    mesh=_mesh,
    scratch_types=[
        pltpu.VMEM((CH_BIG, K), jnp.int32),
        pltpu.VMEM((CH_BIG, K), jnp.int32),
        pltpu.VMEM((K, D), jnp.float32),
        pltpu.SemaphoreType.DMA,
        pltpu.VMEM_SHARED((N_PAD, D), jnp.float32),
    ],
)


def _sc_count_body(dst_hbm, zeros_hbm, ones_hbm, cnt_hbm, dst_v, ones_v,
                   acc):
    c = lax.axis_index("c")
    s = lax.axis_index("s")
    wid = s * NC + c
    mych = jnp.where(c == BIG_CORE, CH_BIG, CH_SMALL)
    slab = pl.ds(s * SLAB, SLAB)
    pltpu.sync_copy(zeros_hbm.at[slab], acc.at[slab])
    pltpu.sync_copy(ones_hbm, ones_v)
    pltpu.sync_copy(dst_hbm.at[wid], dst_v)
    plsc.subcore_barrier()

    def cbody(j, carry):
        pltpu.sync_copy(ones_v, acc.at[dst_v.at[j]], add=True)
        return carry

    lax.fori_loop(0, mych, cbody, 0)
    plsc.subcore_barrier()
    pltpu.sync_copy(acc.at[slab], cnt_hbm.at[c, slab])


_sc_count = pl.kernel(
    _sc_count_body,
    out_type=_sc_out,
    mesh=_mesh,
    scratch_types=[
        pltpu.VMEM((CH_BIG, K), jnp.int32),
        pltpu.VMEM((K, D), jnp.float32),
        pltpu.VMEM_SHARED((N_PAD, D), jnp.float32),
    ],
)


BR = 512


def _tc_dense_body(sums_ref, h_ref, cnt_ref, wa_ref, wh_ref, b_ref, out_ref):
    ssum = sums_ref[0] + sums_ref[1]
    cnt = cnt_ref[0][:, :1] + cnt_ref[1][:, :1]
    inv = 1.0 / jnp.maximum(cnt, 1.0)
    agg = jnp.dot(ssum, wa_ref[...], preferred_element_type=jnp.float32) * inv
    o = agg + jnp.dot(h_ref[...], wh_ref[...],
                      preferred_element_type=jnp.float32) + b_ref[...]
    out_ref[...] = jnp.tanh(o)


_tc_dense = pl.pallas_call(
    _tc_dense_body,
    grid=(N_PAD // BR,),
    in_specs=[
        pl.BlockSpec((NC, BR, D), lambda i: (0, i, 0)),
        pl.BlockSpec((BR, D), lambda i: (i, 0)),
        pl.BlockSpec((NC, BR, D), lambda i: (0, i, 0)),
        pl.BlockSpec((D, D), lambda i: (0, 0)),
        pl.BlockSpec((D, D), lambda i: (0, 0)),
        pl.BlockSpec((1, D), lambda i: (0, 0)),
    ],
    out_specs=pl.BlockSpec((BR, D), lambda i: (i, 0)),
    out_shape=jax.ShapeDtypeStruct((N_PAD, D), jnp.float32),
)


def kernel(x, edge_index, W1, b1, W2, b2, W3, b3):
    src = edge_index[0].astype(jnp.int32)
    dst = edge_index[1].astype(jnp.int32)
    pad = NS * E_TILE_PAIR - E
    cap_big = NS * CH_BIG * K

    def _layout(flat):
        big = flat[:cap_big].reshape(NS, CH_BIG, K)
        small = flat[cap_big:].reshape(NS, CH_SMALL, K)
        small = jnp.concatenate(
            [small, jnp.zeros((NS, CH_BIG - CH_SMALL, K), jnp.int32)], axis=1)
        pair = (big, small) if BIG_CORE == 0 else (small, big)
        return jnp.stack(pair, axis=1).reshape(NW, CH_BIG, K)

    src3 = _layout(jnp.concatenate([src, jnp.zeros((pad,), jnp.int32)]))
    dst3 = _layout(jnp.concatenate([dst, jnp.full((pad,), SCRAP, jnp.int32)]))
    zeros = jnp.zeros((N_PAD, D), jnp.float32)
    ones_c = jnp.ones((K, D), jnp.float32)
    h = jnp.zeros((N_PAD, D), jnp.float32).at[:N].set(x)

    cnt2 = _sc_count(dst3, zeros, ones_c)
    for W, b in ((W1, b1), (W2, b2), (W3, b3)):
        waT = W[:, :D].T
        whT = W[:, D:].T
        sums2 = _sc_scatter(h, src3, dst3, zeros)
        h = _tc_dense(sums2, h, cnt2, waT, whT, b.reshape(1, D))
    return h[:N]
